# baseline (speedup 1.0000x reference)
import functools

import jax
import jax.numpy as jnp
from jax import lax
from jax.experimental import pallas as pl
from jax.experimental.pallas import tpu as pltpu
from jax.experimental.pallas import tpu_sc as plsc

B = 8
CIN = 8
L = 4096
T1 = 2048
T2 = 1024
C1 = 32
ENC = 256
K = 1024
G3 = 3 * ENC

HI = lax.Precision.HIGHEST


def _mm(a, b, prec=HI):
    return lax.dot_general(a, b, (((1,), (0,)), ((), ())),
                           precision=prec, preferred_element_type=jnp.float32)


def _mm_bf16(a, b):
    return lax.dot_general(a.astype(jnp.bfloat16), b.astype(jnp.bfloat16),
                           (((1,), (0,)), ((), ())),
                           preferred_element_type=jnp.float32)


def _conv1_kernel(a_ref, w_ref, b_ref, o_ref):
    o_ref[...] = jnp.maximum(
        _mm_bf16(a_ref[...], w_ref[...]) + b_ref[...], 0.0
    ).astype(jnp.bfloat16)


def _conv1(a1, w1, b1):
    return pl.pallas_call(
        _conv1_kernel,
        out_shape=jax.ShapeDtypeStruct((B * T1, C1), jnp.bfloat16),
    )(a1, w1, b1)


VQ_ROWS = 2048


def _vq_kernel(a_ref, w_ref, b_ref, cbt_ref, codes_ref):
    z = jnp.maximum(_mm_bf16(a_ref[...], w_ref[...]) + b_ref[...], 0.0)
    dots = _mm_bf16(z, cbt_ref[...])
    cnorm = jnp.sum(cbt_ref[...] * cbt_ref[...], axis=0)
    lnorm = jnp.sum(z * z, axis=1, keepdims=True)
    dist = (lnorm + cnorm[None, :]) - 2.0 * dots
    m = jnp.min(dist, axis=1, keepdims=True)
    iota = lax.broadcasted_iota(jnp.int32, dist.shape, 1)
    codes = jnp.min(jnp.where(dist == m, iota, K), axis=1)
    codes_ref[0, 0, :] = codes.astype(jnp.int32)


def _vq(a2, w2, b2, cbt):
    nblk = (B * T2) // VQ_ROWS
    codes3 = pl.pallas_call(
        _vq_kernel,
        grid=(nblk,),
        in_specs=[
            pl.BlockSpec((VQ_ROWS, 5 * C1), lambda i: (i, 0)),
            pl.BlockSpec((5 * C1, ENC), lambda i: (0, 0)),
            pl.BlockSpec((1, ENC), lambda i: (0, 0)),
            pl.BlockSpec((ENC, K), lambda i: (0, 0)),
        ],
        out_specs=pl.BlockSpec((1, 1, VQ_ROWS), lambda i: (i, 0, 0)),
        out_shape=jax.ShapeDtypeStruct((nblk, 1, VQ_ROWS), jnp.int32),
    )(a2, w2, b2, cbt)
    return codes3.reshape(B * T2)


def _sc_gather(table, idx):
    info = plsc.get_sparse_core_info()
    nc, ns = info.num_cores, info.num_subcores
    nw = nc * ns
    n = idx.shape[0]
    b_per_w = n // nw
    mesh = plsc.VectorSubcoreMesh(core_axis_name="c", subcore_axis_name="s")

    @functools.partial(
        pl.kernel, mesh=mesh,
        out_type=jax.ShapeDtypeStruct((n, ENC), jnp.float32),
        scratch_types=[
            pltpu.VMEM((b_per_w,), jnp.int32),
            pltpu.VMEM((b_per_w, ENC), jnp.float32),
            pltpu.SemaphoreType.DMA,
        ],
    )
    def gather_k(table_hbm, idx_hbm, out_hbm, idx_v, rows_v, sem):
        wid = lax.axis_index("s") * nc + lax.axis_index("c")
        base = wid * b_per_w
        pltpu.sync_copy(idx_hbm.at[pl.ds(base, b_per_w)], idx_v)
        pltpu.async_copy(table_hbm.at[idx_v], rows_v, sem).wait()
        pltpu.sync_copy(rows_v, out_hbm.at[pl.ds(base, b_per_w)])

    return gather_k(table, idx)


CH = 128


def _gru_kernel(q_ref, wih_ref, bih_ref, whh_ref, bhh_ref, pw_ref, pb_ref,
                pred_ref, ha_ref, hb_ref, gx_ref, ctx_ref):
    @pl.when(pl.program_id(0) == 0)
    def _():
        ha_ref[...] = jnp.zeros_like(ha_ref)
        hb_ref[...] = jnp.zeros_like(hb_ref)

    gx_ref[...] = _mm_bf16(q_ref[...], wih_ref[...]) + bih_ref[...]

    HB = B // 2

    def gates(gx, gh, h):
        r = jax.nn.sigmoid(gx[:, :ENC] + gh[:, :ENC])
        zt = jax.nn.sigmoid(gx[:, ENC:2 * ENC] + gh[:, ENC:2 * ENC])
        n = jnp.tanh(gx[:, 2 * ENC:] + r * gh[:, 2 * ENC:])
        return (1.0 - zt) * n + zt * h

    def body(t, _):
        base = t * B
        ha = ha_ref[...]
        hb = hb_ref[...]
        gha = _mm_bf16(ha, whh_ref[...]) + bhh_ref[...]
        ghb = _mm_bf16(hb, whh_ref[...]) + bhh_ref[...]
        gx_t = gx_ref[pl.ds(base, B), :]
        hna = gates(gx_t[:HB], gha, ha)
        hnb = gates(gx_t[HB:], ghb, hb)
        ha_ref[...] = hna
        hb_ref[...] = hnb
        ctx_ref[pl.ds(base, B), :] = jnp.concatenate([hna, hnb], axis=0)
        return 0

    lax.fori_loop(0, CH, body, 0, unroll=2)
    pred_ref[...] = _mm_bf16(ctx_ref[...], pw_ref[...]) + pb_ref[...]


def _gru(q_tm, wih, bih, whh, bhh, pw, pb):
    nblk = T2 // CH
    rows = CH * B
    return pl.pallas_call(
        _gru_kernel,
        grid=(nblk,),
        in_specs=[
            pl.BlockSpec((rows, ENC), lambda i: (i, 0)),
            pl.BlockSpec((ENC, G3), lambda i: (0, 0)),
            pl.BlockSpec((1, G3), lambda i: (0, 0)),
            pl.BlockSpec((ENC, G3), lambda i: (0, 0)),
            pl.BlockSpec((1, G3), lambda i: (0, 0)),
            pl.BlockSpec((ENC, ENC), lambda i: (0, 0)),
            pl.BlockSpec((1, ENC), lambda i: (0, 0)),
        ],
        out_specs=pl.BlockSpec((rows, ENC), lambda i: (i, 0)),
        out_shape=jax.ShapeDtypeStruct((B * T2, ENC), jnp.float32),
        scratch_shapes=[
            pltpu.VMEM((B // 2, ENC), jnp.float32),
            pltpu.VMEM((B // 2, ENC), jnp.float32),
            pltpu.VMEM((rows, G3), jnp.float32),
            pltpu.VMEM((rows, ENC), jnp.float32),
        ],
    )(q_tm, wih, bih, whh, bhh, pw, pb)


def kernel(x, conv1_w, conv1_b, conv2_w, conv2_b, codebook,
           gru_w_ih, gru_w_hh, gru_b_ih, gru_b_hh, proj_w, proj_b):
    x_pad = jnp.pad(x.astype(jnp.bfloat16), ((0, 0), (0, 0), (2, 2)))
    a1 = jnp.stack([x_pad[:, :, k:k + 2 * T1:2] for k in range(5)], axis=-1)
    a1 = a1.transpose(0, 2, 1, 3).reshape(B * T1, CIN * 5)
    w1 = conv1_w.transpose(1, 2, 0).reshape(CIN * 5, C1)
    z1 = _conv1(a1, w1, conv1_b.reshape(1, C1)).reshape(B, T1, C1)

    z1p = jnp.pad(z1, ((0, 0), (2, 2), (0, 0)))
    a2 = jnp.stack([z1p[:, k:k + 2 * T2:2, :] for k in range(5)], axis=2)
    a2 = a2.reshape(B * T2, 5 * C1)
    w2 = conv2_w.transpose(2, 1, 0).reshape(5 * C1, ENC)

    codes = _vq(a2, w2, conv2_b.reshape(1, ENC), codebook.T)
    codes_bt = codes.reshape(B, T2)

    idx_tm = codes_bt.T.reshape(-1)
    quant_tm = _sc_gather(codebook, idx_tm)
    quant_z = quant_tm.reshape(T2, B, ENC).transpose(1, 0, 2)

    bf = jnp.bfloat16
    pred_tm = _gru(quant_tm, gru_w_ih.T.astype(bf), gru_b_ih.reshape(1, G3),
                   gru_w_hh.T.astype(bf), gru_b_hh.reshape(1, G3),
                   proj_w.T.astype(bf), proj_b.reshape(1, ENC))
    pred = pred_tm.reshape(T2, B, ENC).transpose(1, 0, 2)

    return quant_z, pred, codes_bt


# speedup vs baseline: 1.0546x; 1.0546x over previous
import functools

import jax
import jax.numpy as jnp
from jax import lax
from jax.experimental import pallas as pl
from jax.experimental.pallas import tpu as pltpu
from jax.experimental.pallas import tpu_sc as plsc

B = 8
CIN = 8
L = 4096
T1 = 2048
T2 = 1024
C1 = 32
ENC = 256
K = 1024
G3 = 3 * ENC

HI = lax.Precision.HIGHEST


def _mm(a, b, prec=HI):
    return lax.dot_general(a, b, (((1,), (0,)), ((), ())),
                           precision=prec, preferred_element_type=jnp.float32)


def _mm_bf16(a, b):
    return lax.dot_general(a.astype(jnp.bfloat16), b.astype(jnp.bfloat16),
                           (((1,), (0,)), ((), ())),
                           preferred_element_type=jnp.float32)


def _conv1_kernel(a_ref, w_ref, b_ref, o_ref):
    o_ref[...] = jnp.maximum(
        _mm_bf16(a_ref[...], w_ref[...]) + b_ref[...], 0.0
    ).astype(jnp.bfloat16)


def _conv1(a1, w1, b1):
    return pl.pallas_call(
        _conv1_kernel,
        out_shape=jax.ShapeDtypeStruct((B * T1, C1), jnp.bfloat16),
    )(a1, w1, b1)


VQ_ROWS = 2048


def _vq_kernel(a_ref, w_ref, b_ref, cbt_ref, codes_ref):
    z = jnp.maximum(_mm_bf16(a_ref[...], w_ref[...]) + b_ref[...], 0.0)
    dots = _mm_bf16(z, cbt_ref[...])
    cnorm = jnp.sum(cbt_ref[...] * cbt_ref[...], axis=0)
    lnorm = jnp.sum(z * z, axis=1, keepdims=True)
    dist = (lnorm + cnorm[None, :]) - 2.0 * dots
    m = jnp.min(dist, axis=1, keepdims=True)
    iota = lax.broadcasted_iota(jnp.int32, dist.shape, 1)
    codes = jnp.min(jnp.where(dist == m, iota, K), axis=1)
    codes_ref[0, 0, :] = codes.astype(jnp.int32)


def _vq(a2, w2, b2, cbt):
    nblk = (B * T2) // VQ_ROWS
    codes3 = pl.pallas_call(
        _vq_kernel,
        grid=(nblk,),
        in_specs=[
            pl.BlockSpec((VQ_ROWS, 5 * C1), lambda i: (i, 0)),
            pl.BlockSpec((5 * C1, ENC), lambda i: (0, 0)),
            pl.BlockSpec((1, ENC), lambda i: (0, 0)),
            pl.BlockSpec((ENC, K), lambda i: (0, 0)),
        ],
        out_specs=pl.BlockSpec((1, 1, VQ_ROWS), lambda i: (i, 0, 0)),
        out_shape=jax.ShapeDtypeStruct((nblk, 1, VQ_ROWS), jnp.int32),
    )(a2, w2, b2, cbt)
    return codes3.reshape(B * T2)


def _sc_gather(table, idx):
    info = plsc.get_sparse_core_info()
    nc, ns = info.num_cores, info.num_subcores
    nw = nc * ns
    n = idx.shape[0]
    b_per_w = n // nw
    mesh = plsc.VectorSubcoreMesh(core_axis_name="c", subcore_axis_name="s")

    @functools.partial(
        pl.kernel, mesh=mesh,
        out_type=jax.ShapeDtypeStruct((n, ENC), jnp.float32),
        scratch_types=[
            pltpu.VMEM((b_per_w,), jnp.int32),
            pltpu.VMEM((b_per_w, ENC), jnp.float32),
            pltpu.SemaphoreType.DMA,
        ],
    )
    def gather_k(table_hbm, idx_hbm, out_hbm, idx_v, rows_v, sem):
        wid = lax.axis_index("s") * nc + lax.axis_index("c")
        base = wid * b_per_w
        pltpu.sync_copy(idx_hbm.at[pl.ds(base, b_per_w)], idx_v)
        pltpu.async_copy(table_hbm.at[idx_v], rows_v, sem).wait()
        pltpu.sync_copy(rows_v, out_hbm.at[pl.ds(base, b_per_w)])

    return gather_k(table, idx)


CH = 128


def _gru_kernel(q_ref, wih_ref, bih_ref, whh_ref, bhh_ref, pw_ref, pb_ref,
                pred_ref, ha_ref, gx_ref, ctx_ref):
    @pl.when(pl.program_id(0) == 0)
    def _():
        ha_ref[...] = jnp.zeros_like(ha_ref)

    gx_ref[...] = _mm_bf16(q_ref[...], wih_ref[...]) + bih_ref[...]

    def gates(gx, gh, h):
        r = jax.nn.sigmoid(gx[:, :ENC] + gh[:, :ENC])
        zt = jax.nn.sigmoid(gx[:, ENC:2 * ENC] + gh[:, ENC:2 * ENC])
        n = jnp.tanh(gx[:, 2 * ENC:] + r * gh[:, 2 * ENC:])
        return (1.0 - zt) * n + zt * h

    def body(t, _):
        base = t * B
        h = ha_ref[...]
        gh = _mm_bf16(h, whh_ref[...]) + bhh_ref[...]
        hn = gates(gx_ref[pl.ds(base, B), :], gh, h)
        ha_ref[...] = hn
        ctx_ref[pl.ds(base, B), :] = hn
        return 0

    lax.fori_loop(0, CH, body, 0, unroll=2)
    pred_ref[...] = _mm_bf16(ctx_ref[...], pw_ref[...]) + pb_ref[...]


def _gru(q_tm, wih, bih, whh, bhh, pw, pb):
    nblk = T2 // CH
    rows = CH * B
    return pl.pallas_call(
        _gru_kernel,
        grid=(nblk,),
        in_specs=[
            pl.BlockSpec((rows, ENC), lambda i: (i, 0)),
            pl.BlockSpec((ENC, G3), lambda i: (0, 0)),
            pl.BlockSpec((1, G3), lambda i: (0, 0)),
            pl.BlockSpec((ENC, G3), lambda i: (0, 0)),
            pl.BlockSpec((1, G3), lambda i: (0, 0)),
            pl.BlockSpec((ENC, ENC), lambda i: (0, 0)),
            pl.BlockSpec((1, ENC), lambda i: (0, 0)),
        ],
        out_specs=pl.BlockSpec((rows, ENC), lambda i: (i, 0)),
        out_shape=jax.ShapeDtypeStruct((B * T2, ENC), jnp.float32),
        scratch_shapes=[
            pltpu.VMEM((B, ENC), jnp.float32),
            pltpu.VMEM((rows, G3), jnp.float32),
            pltpu.VMEM((rows, ENC), jnp.float32),
        ],
    )(q_tm, wih, bih, whh, bhh, pw, pb)


def kernel(x, conv1_w, conv1_b, conv2_w, conv2_b, codebook,
           gru_w_ih, gru_w_hh, gru_b_ih, gru_b_hh, proj_w, proj_b):
    x_pad = jnp.pad(x.astype(jnp.bfloat16), ((0, 0), (0, 0), (2, 2)))
    a1 = jnp.stack([x_pad[:, :, k:k + 2 * T1:2] for k in range(5)], axis=-1)
    a1 = a1.transpose(0, 2, 1, 3).reshape(B * T1, CIN * 5)
    w1 = conv1_w.transpose(1, 2, 0).reshape(CIN * 5, C1)
    z1 = _conv1(a1, w1, conv1_b.reshape(1, C1)).reshape(B, T1, C1)

    z1p = jnp.pad(z1, ((0, 0), (2, 2), (0, 0)))
    a2 = jnp.stack([z1p[:, k:k + 2 * T2:2, :] for k in range(5)], axis=2)
    a2 = a2.reshape(B * T2, 5 * C1)
    w2 = conv2_w.transpose(2, 1, 0).reshape(5 * C1, ENC)

    codes = _vq(a2, w2, conv2_b.reshape(1, ENC), codebook.T)
    codes_bt = codes.reshape(B, T2)

    idx_tm = codes_bt.T.reshape(-1)
    quant_tm = _sc_gather(codebook, idx_tm)
    quant_z = quant_tm.reshape(T2, B, ENC).transpose(1, 0, 2)

    bf = jnp.bfloat16
    pred_tm = _gru(quant_tm, gru_w_ih.T.astype(bf), gru_b_ih.reshape(1, G3),
                   gru_w_hh.T.astype(bf), gru_b_hh.reshape(1, G3),
                   proj_w.T.astype(bf), proj_b.reshape(1, ENC))
    pred = pred_tm.reshape(T2, B, ENC).transpose(1, 0, 2)

    return quant_z, pred, codes_bt


# speedup vs baseline: 1.0701x; 1.0147x over previous
import functools

import jax
import jax.numpy as jnp
from jax import lax
from jax.experimental import pallas as pl
from jax.experimental.pallas import tpu as pltpu
from jax.experimental.pallas import tpu_sc as plsc

B = 8
CIN = 8
L = 4096
T1 = 2048
T2 = 1024
C1 = 32
ENC = 256
K = 1024
G3 = 3 * ENC

HI = lax.Precision.HIGHEST


def _mm(a, b, prec=HI):
    return lax.dot_general(a, b, (((1,), (0,)), ((), ())),
                           precision=prec, preferred_element_type=jnp.float32)


def _mm_bf16(a, b):
    return lax.dot_general(a.astype(jnp.bfloat16), b.astype(jnp.bfloat16),
                           (((1,), (0,)), ((), ())),
                           preferred_element_type=jnp.float32)


def _conv1_kernel(a_ref, w_ref, b_ref, o_ref):
    o_ref[...] = jnp.maximum(
        _mm_bf16(a_ref[...], w_ref[...]) + b_ref[...], 0.0
    ).astype(jnp.bfloat16)


def _conv1(a1, w1, b1):
    return pl.pallas_call(
        _conv1_kernel,
        out_shape=jax.ShapeDtypeStruct((B * T1, C1), jnp.bfloat16),
    )(a1, w1, b1)


VQ_ROWS = 2048


def _vq_kernel(a_ref, w_ref, b_ref, cbt_ref, codes_ref):
    z = jnp.maximum(_mm_bf16(a_ref[...], w_ref[...]) + b_ref[...], 0.0)
    dots = _mm_bf16(z, cbt_ref[...])
    cnorm = jnp.sum(cbt_ref[...] * cbt_ref[...], axis=0)
    lnorm = jnp.sum(z * z, axis=1, keepdims=True)
    dist = (lnorm + cnorm[None, :]) - 2.0 * dots
    m = jnp.min(dist, axis=1, keepdims=True)
    iota = lax.broadcasted_iota(jnp.int32, dist.shape, 1)
    codes = jnp.min(jnp.where(dist == m, iota, K), axis=1)
    codes_ref[0, 0, :] = codes.astype(jnp.int32)


def _vq(a2, w2, b2, cbt):
    nblk = (B * T2) // VQ_ROWS
    codes3 = pl.pallas_call(
        _vq_kernel,
        grid=(nblk,),
        in_specs=[
            pl.BlockSpec((VQ_ROWS, 5 * C1), lambda i: (i, 0)),
            pl.BlockSpec((5 * C1, ENC), lambda i: (0, 0)),
            pl.BlockSpec((1, ENC), lambda i: (0, 0)),
            pl.BlockSpec((ENC, K), lambda i: (0, 0)),
        ],
        out_specs=pl.BlockSpec((1, 1, VQ_ROWS), lambda i: (i, 0, 0)),
        out_shape=jax.ShapeDtypeStruct((nblk, 1, VQ_ROWS), jnp.int32),
    )(a2, w2, b2, cbt)
    return codes3.reshape(B * T2)


def _sc_gather(table, idx):
    info = plsc.get_sparse_core_info()
    nc, ns = info.num_cores, info.num_subcores
    nw = nc * ns
    n = idx.shape[0]
    b_per_w = n // nw
    mesh = plsc.VectorSubcoreMesh(core_axis_name="c", subcore_axis_name="s")

    h = b_per_w // 2

    @functools.partial(
        pl.kernel, mesh=mesh,
        out_type=jax.ShapeDtypeStruct((n, ENC), jnp.float32),
        scratch_types=[
            pltpu.VMEM((h,), jnp.int32),
            pltpu.VMEM((h,), jnp.int32),
            pltpu.VMEM((h, ENC), jnp.float32),
            pltpu.VMEM((h, ENC), jnp.float32),
            pltpu.SemaphoreType.DMA,
            pltpu.SemaphoreType.DMA,
            pltpu.SemaphoreType.DMA,
            pltpu.SemaphoreType.DMA,
        ],
    )
    def gather_k(table_hbm, idx_hbm, out_hbm,
                 idx0, idx1, r0, r1, s0, s1, o0, o1):
        wid = lax.axis_index("s") * nc + lax.axis_index("c")
        base = wid * b_per_w
        pltpu.sync_copy(idx_hbm.at[pl.ds(base, h)], idx0)
        pltpu.sync_copy(idx_hbm.at[pl.ds(base + h, h)], idx1)
        c0 = pltpu.async_copy(table_hbm.at[idx0], r0, s0)
        c1 = pltpu.async_copy(table_hbm.at[idx1], r1, s1)
        c0.wait()
        w0 = pltpu.async_copy(r0, out_hbm.at[pl.ds(base, h)], o0)
        c1.wait()
        w1 = pltpu.async_copy(r1, out_hbm.at[pl.ds(base + h, h)], o1)
        w0.wait()
        w1.wait()

    return gather_k(table, idx)


CH = 128


def _gru_kernel(q_ref, wih_ref, bih_ref, whh_ref, bhh_ref, pw_ref, pb_ref,
                pred_ref, ha_ref, gx_ref, ctx_ref):
    @pl.when(pl.program_id(0) == 0)
    def _():
        ha_ref[...] = jnp.zeros_like(ha_ref)

    gx_ref[...] = _mm_bf16(q_ref[...], wih_ref[...]) + bih_ref[...]

    def gates(gx, gh, h):
        r = jax.nn.sigmoid(gx[:, :ENC] + gh[:, :ENC])
        zt = jax.nn.sigmoid(gx[:, ENC:2 * ENC] + gh[:, ENC:2 * ENC])
        n = jnp.tanh(gx[:, 2 * ENC:] + r * gh[:, 2 * ENC:])
        return (1.0 - zt) * n + zt * h

    def body(t, _):
        base = t * B
        h = ha_ref[...]
        gh = _mm_bf16(h, whh_ref[...]) + bhh_ref[...]
        hn = gates(gx_ref[pl.ds(base, B), :], gh, h)
        ha_ref[...] = hn
        ctx_ref[pl.ds(base, B), :] = hn
        return 0

    lax.fori_loop(0, CH, body, 0, unroll=4)
    pred_ref[...] = _mm_bf16(ctx_ref[...], pw_ref[...]) + pb_ref[...]


def _gru(q_tm, wih, bih, whh, bhh, pw, pb):
    nblk = T2 // CH
    rows = CH * B
    return pl.pallas_call(
        _gru_kernel,
        grid=(nblk,),
        in_specs=[
            pl.BlockSpec((rows, ENC), lambda i: (i, 0)),
            pl.BlockSpec((ENC, G3), lambda i: (0, 0)),
            pl.BlockSpec((1, G3), lambda i: (0, 0)),
            pl.BlockSpec((ENC, G3), lambda i: (0, 0)),
            pl.BlockSpec((1, G3), lambda i: (0, 0)),
            pl.BlockSpec((ENC, ENC), lambda i: (0, 0)),
            pl.BlockSpec((1, ENC), lambda i: (0, 0)),
        ],
        out_specs=pl.BlockSpec((rows, ENC), lambda i: (i, 0)),
        out_shape=jax.ShapeDtypeStruct((B * T2, ENC), jnp.float32),
        scratch_shapes=[
            pltpu.VMEM((B, ENC), jnp.float32),
            pltpu.VMEM((rows, G3), jnp.float32),
            pltpu.VMEM((rows, ENC), jnp.float32),
        ],
    )(q_tm, wih, bih, whh, bhh, pw, pb)


def kernel(x, conv1_w, conv1_b, conv2_w, conv2_b, codebook,
           gru_w_ih, gru_w_hh, gru_b_ih, gru_b_hh, proj_w, proj_b):
    x_pad = jnp.pad(x.astype(jnp.bfloat16), ((0, 0), (0, 0), (2, 2)))
    a1 = jnp.stack([x_pad[:, :, k:k + 2 * T1:2] for k in range(5)], axis=-1)
    a1 = a1.transpose(0, 2, 1, 3).reshape(B * T1, CIN * 5)
    w1 = conv1_w.transpose(1, 2, 0).reshape(CIN * 5, C1)
    z1 = _conv1(a1, w1, conv1_b.reshape(1, C1)).reshape(B, T1, C1)

    z1p = jnp.pad(z1, ((0, 0), (2, 2), (0, 0)))
    a2 = jnp.stack([z1p[:, k:k + 2 * T2:2, :] for k in range(5)], axis=2)
    a2 = a2.reshape(B * T2, 5 * C1)
    w2 = conv2_w.transpose(2, 1, 0).reshape(5 * C1, ENC)

    codes = _vq(a2, w2, conv2_b.reshape(1, ENC), codebook.T)
    codes_bt = codes.reshape(B, T2)

    idx_tm = codes_bt.T.reshape(-1)
    quant_tm = _sc_gather(codebook, idx_tm)
    quant_z = quant_tm.reshape(T2, B, ENC).transpose(1, 0, 2)

    bf = jnp.bfloat16
    pred_tm = _gru(quant_tm, gru_w_ih.T.astype(bf), gru_b_ih.reshape(1, G3),
                   gru_w_hh.T.astype(bf), gru_b_hh.reshape(1, G3),
                   proj_w.T.astype(bf), proj_b.reshape(1, ENC))
    pred = pred_tm.reshape(T2, B, ENC).transpose(1, 0, 2)

    return quant_z, pred, codes_bt


# speedup vs baseline: 1.0720x; 1.0018x over previous
import functools

import jax
import jax.numpy as jnp
from jax import lax
from jax.experimental import pallas as pl
from jax.experimental.pallas import tpu as pltpu
from jax.experimental.pallas import tpu_sc as plsc

B = 8
CIN = 8
L = 4096
T1 = 2048
T2 = 1024
C1 = 32
ENC = 256
K = 1024
G3 = 3 * ENC

HI = lax.Precision.HIGHEST


def _mm(a, b, prec=HI):
    return lax.dot_general(a, b, (((1,), (0,)), ((), ())),
                           precision=prec, preferred_element_type=jnp.float32)


def _mm_bf16(a, b):
    return lax.dot_general(a.astype(jnp.bfloat16), b.astype(jnp.bfloat16),
                           (((1,), (0,)), ((), ())),
                           preferred_element_type=jnp.float32)


def _conv1_kernel(a_ref, w_ref, b_ref, o_ref):
    o_ref[...] = jnp.maximum(
        _mm_bf16(a_ref[...], w_ref[...]) + b_ref[...], 0.0
    ).astype(jnp.bfloat16)


def _conv1(a1, w1, b1):
    return pl.pallas_call(
        _conv1_kernel,
        out_shape=jax.ShapeDtypeStruct((B * T1, C1), jnp.bfloat16),
    )(a1, w1, b1)


VQ_ROWS = 2048


def _vq_kernel(a_ref, w_ref, b_ref, cbt_ref, codes_ref):
    z = jnp.maximum(_mm_bf16(a_ref[...], w_ref[...]) + b_ref[...], 0.0)
    dots = _mm_bf16(z, cbt_ref[...])
    cnorm = jnp.sum(cbt_ref[...] * cbt_ref[...], axis=0)
    lnorm = jnp.sum(z * z, axis=1, keepdims=True)
    dist = (lnorm + cnorm[None, :]) - 2.0 * dots
    m = jnp.min(dist, axis=1, keepdims=True)
    iota = lax.broadcasted_iota(jnp.int32, dist.shape, 1)
    codes = jnp.min(jnp.where(dist == m, iota, K), axis=1)
    codes_ref[0, 0, :] = codes.astype(jnp.int32)


def _vq(a2, w2, b2, cbt):
    nblk = (B * T2) // VQ_ROWS
    codes3 = pl.pallas_call(
        _vq_kernel,
        grid=(nblk,),
        in_specs=[
            pl.BlockSpec((VQ_ROWS, 5 * C1), lambda i: (i, 0)),
            pl.BlockSpec((5 * C1, ENC), lambda i: (0, 0)),
            pl.BlockSpec((1, ENC), lambda i: (0, 0)),
            pl.BlockSpec((ENC, K), lambda i: (0, 0)),
        ],
        out_specs=pl.BlockSpec((1, 1, VQ_ROWS), lambda i: (i, 0, 0)),
        out_shape=jax.ShapeDtypeStruct((nblk, 1, VQ_ROWS), jnp.int32),
    )(a2, w2, b2, cbt)
    return codes3.reshape(B * T2)


def _sc_gather(table, idx):
    info = plsc.get_sparse_core_info()
    nc, ns = info.num_cores, info.num_subcores
    nw = nc * ns
    n = idx.shape[0]
    b_per_w = n // nw
    mesh = plsc.VectorSubcoreMesh(core_axis_name="c", subcore_axis_name="s")

    h = b_per_w // 2

    @functools.partial(
        pl.kernel, mesh=mesh,
        out_type=jax.ShapeDtypeStruct((n, ENC), jnp.float32),
        scratch_types=[
            pltpu.VMEM((h,), jnp.int32),
            pltpu.VMEM((h,), jnp.int32),
            pltpu.VMEM((h, ENC), jnp.float32),
            pltpu.VMEM((h, ENC), jnp.float32),
            pltpu.SemaphoreType.DMA,
            pltpu.SemaphoreType.DMA,
            pltpu.SemaphoreType.DMA,
            pltpu.SemaphoreType.DMA,
        ],
    )
    def gather_k(table_hbm, idx_hbm, out_hbm,
                 idx0, idx1, r0, r1, s0, s1, o0, o1):
        wid = lax.axis_index("s") * nc + lax.axis_index("c")
        base = wid * b_per_w
        pltpu.sync_copy(idx_hbm.at[pl.ds(base, h)], idx0)
        pltpu.sync_copy(idx_hbm.at[pl.ds(base + h, h)], idx1)
        c0 = pltpu.async_copy(table_hbm.at[idx0], r0, s0)
        c1 = pltpu.async_copy(table_hbm.at[idx1], r1, s1)
        c0.wait()
        w0 = pltpu.async_copy(r0, out_hbm.at[pl.ds(base, h)], o0)
        c1.wait()
        w1 = pltpu.async_copy(r1, out_hbm.at[pl.ds(base + h, h)], o1)
        w0.wait()
        w1.wait()

    return gather_k(table, idx)


CH = 256


def _gru_kernel(q_ref, wih_ref, bih_ref, whh_ref, bhh_ref, pw_ref, pb_ref,
                pred_ref, ha_ref, gx_ref, ctx_ref):
    @pl.when(pl.program_id(0) == 0)
    def _():
        ha_ref[...] = jnp.zeros_like(ha_ref)

    gx_ref[...] = _mm_bf16(q_ref[...], wih_ref[...]) + bih_ref[...]

    def gates(gx, gh, h):
        r = jax.nn.sigmoid(gx[:, :ENC] + gh[:, :ENC])
        zt = jax.nn.sigmoid(gx[:, ENC:2 * ENC] + gh[:, ENC:2 * ENC])
        n = jnp.tanh(gx[:, 2 * ENC:] + r * gh[:, 2 * ENC:])
        return (1.0 - zt) * n + zt * h

    def body(t, _):
        base = t * B
        h = ha_ref[...]
        gh = _mm_bf16(h, whh_ref[...]) + bhh_ref[...]
        hn = gates(gx_ref[pl.ds(base, B), :], gh, h)
        ha_ref[...] = hn
        ctx_ref[pl.ds(base, B), :] = hn
        return 0

    lax.fori_loop(0, CH, body, 0, unroll=4)
    pred_ref[...] = _mm_bf16(ctx_ref[...], pw_ref[...]) + pb_ref[...]


def _gru(q_tm, wih, bih, whh, bhh, pw, pb):
    nblk = T2 // CH
    rows = CH * B
    return pl.pallas_call(
        _gru_kernel,
        grid=(nblk,),
        in_specs=[
            pl.BlockSpec((rows, ENC), lambda i: (i, 0)),
            pl.BlockSpec((ENC, G3), lambda i: (0, 0)),
            pl.BlockSpec((1, G3), lambda i: (0, 0)),
            pl.BlockSpec((ENC, G3), lambda i: (0, 0)),
            pl.BlockSpec((1, G3), lambda i: (0, 0)),
            pl.BlockSpec((ENC, ENC), lambda i: (0, 0)),
            pl.BlockSpec((1, ENC), lambda i: (0, 0)),
        ],
        out_specs=pl.BlockSpec((rows, ENC), lambda i: (i, 0)),
        out_shape=jax.ShapeDtypeStruct((B * T2, ENC), jnp.float32),
        scratch_shapes=[
            pltpu.VMEM((B, ENC), jnp.float32),
            pltpu.VMEM((rows, G3), jnp.float32),
            pltpu.VMEM((rows, ENC), jnp.float32),
        ],
    )(q_tm, wih, bih, whh, bhh, pw, pb)


def kernel(x, conv1_w, conv1_b, conv2_w, conv2_b, codebook,
           gru_w_ih, gru_w_hh, gru_b_ih, gru_b_hh, proj_w, proj_b):
    x_pad = jnp.pad(x.astype(jnp.bfloat16), ((0, 0), (0, 0), (2, 2)))
    a1 = jnp.stack([x_pad[:, :, k:k + 2 * T1:2] for k in range(5)], axis=-1)
    a1 = a1.transpose(0, 2, 1, 3).reshape(B * T1, CIN * 5)
    w1 = conv1_w.transpose(1, 2, 0).reshape(CIN * 5, C1)
    z1 = _conv1(a1, w1, conv1_b.reshape(1, C1)).reshape(B, T1, C1)

    z1p = jnp.pad(z1, ((0, 0), (2, 2), (0, 0)))
    a2 = jnp.stack([z1p[:, k:k + 2 * T2:2, :] for k in range(5)], axis=2)
    a2 = a2.reshape(B * T2, 5 * C1)
    w2 = conv2_w.transpose(2, 1, 0).reshape(5 * C1, ENC)

    codes = _vq(a2, w2, conv2_b.reshape(1, ENC), codebook.T)
    codes_bt = codes.reshape(B, T2)

    idx_tm = codes_bt.T.reshape(-1)
    quant_tm = _sc_gather(codebook, idx_tm)
    quant_z = quant_tm.reshape(T2, B, ENC).transpose(1, 0, 2)

    bf = jnp.bfloat16
    pred_tm = _gru(quant_tm, gru_w_ih.T.astype(bf), gru_b_ih.reshape(1, G3),
                   gru_w_hh.T.astype(bf), gru_b_hh.reshape(1, G3),
                   proj_w.T.astype(bf), proj_b.reshape(1, ENC))
    pred = pred_tm.reshape(T2, B, ENC).transpose(1, 0, 2)

    return quant_z, pred, codes_bt
